# baseline (speedup 1.0000x reference)
import functools
import numpy as np
import jax
import jax.numpy as jnp
from jax import lax
from jax.experimental import pallas as pl
from jax.experimental.pallas import tpu as pltpu
from jax.experimental.pallas import tpu_sc as plsc

DIM = 768
H = 12
KV = 4
HD = 64
HID = 2048
E = 8
T = 2048
BT = 256
QW = H * HD
KW = KV * HD
BTILE = 256
NT = 24
CAP = NT * BTILE


def _halves_perm(nheads):
    perm = []
    for h in range(nheads):
        for i in range(HD // 2):
            perm.append(h * HD + 2 * i)
        for i in range(HD // 2):
            perm.append(h * HD + 2 * i + 1)
    return np.array(perm, dtype=np.int32)


_QPERM = _halves_perm(H)
_KPERM = _halves_perm(KV)


def _qkv_kernel(x_ref, wq_ref, wk_ref, wv_ref, nw_ref,
                cq_ref, sq_ref, ck_ref, sk_ref,
                q_ref, k_ref, v_ref):
    x = x_ref[...]
    rms = jnp.sqrt(jnp.mean(x * x, axis=-1, keepdims=True) + 1e-6)
    h = nw_ref[...] * x / rms
    q = jnp.dot(h, wq_ref[...], preferred_element_type=jnp.float32)
    k = jnp.dot(h, wk_ref[...], preferred_element_type=jnp.float32)
    v = jnp.dot(h, wv_ref[...], preferred_element_type=jnp.float32)
    HH = HD // 2
    qcol = jax.lax.broadcasted_iota(jnp.int32, (BT, QW), 1)
    rq = jnp.where((qcol // HH) % 2 == 0,
                   jnp.roll(q, -HH, axis=1), jnp.roll(q, HH, axis=1))
    kcol = jax.lax.broadcasted_iota(jnp.int32, (BT, KW), 1)
    rk = jnp.where((kcol // HH) % 2 == 0,
                   jnp.roll(k, -HH, axis=1), jnp.roll(k, HH, axis=1))
    q_ref[...] = q * cq_ref[...] + rq * sq_ref[...]
    k_ref[...] = k * ck_ref[...] + rk * sk_ref[...]
    v_ref[...] = v


def _attn_kernel(q_ref, k_ref, v_ref, o_ref):
    i = pl.program_id(0)
    q = q_ref[...]
    scale = HD ** -0.5
    rowid = jax.lax.broadcasted_iota(jnp.int32, (BT, BT), 0)
    colid = jax.lax.broadcasted_iota(jnp.int32, (BT, BT), 1)
    dn = (((1,), (1,)), ((), ()))
    for h in range(H):
        g = h // (H // KV)
        qh = q[:, h * HD:(h + 1) * HD] * scale

        def body(j, carry, g=g, qh=qh):
            m, l, acc = carry
            kc = k_ref[pl.ds(j * BT, BT), g * HD:(g + 1) * HD]
            vc = v_ref[pl.ds(j * BT, BT), g * HD:(g + 1) * HD]
            s = jax.lax.dot_general(qh, kc, dn,
                                    preferred_element_type=jnp.float32)
            s = jnp.where((j * BT + colid) > (i * BT + rowid), -jnp.inf, s)
            m2 = jnp.maximum(m, jnp.max(s, axis=1, keepdims=True))
            p = jnp.exp(s - m2)
            sc = jnp.exp(m - m2)
            l2 = l * sc + jnp.sum(p, axis=1, keepdims=True)
            acc2 = acc * sc + jnp.dot(p, vc,
                                      preferred_element_type=jnp.float32)
            return m2, l2, acc2

        m0 = jnp.full((BT, 1), -jnp.inf, jnp.float32)
        l0 = jnp.zeros((BT, 1), jnp.float32)
        a0 = jnp.zeros((BT, HD), jnp.float32)
        m, l, acc = jax.lax.fori_loop(0, i + 1, body, (m0, l0, a0))
        o_ref[:, h * HD:(h + 1) * HD] = acc / l


def _oproj_router_kernel(x_ref, o_ref, wo_ref, nw_ref, wr_ref, br_ref,
                         x2_ref, h2_ref, g_ref, m_ref):
    x2 = x_ref[...] + jnp.dot(o_ref[...], wo_ref[...],
                              preferred_element_type=jnp.float32)
    x2_ref[...] = x2
    rms = jnp.sqrt(jnp.mean(x2 * x2, axis=-1, keepdims=True) + 1e-6)
    h2 = nw_ref[...] * x2 / rms
    h2_ref[...] = h2
    logits = jnp.dot(h2, wr_ref[...], preferred_element_type=jnp.float32) \
        + br_ref[...]
    m = jnp.max(logits, axis=1, keepdims=True)
    p = jnp.exp(logits - m)
    p = p / jnp.sum(p, axis=1, keepdims=True)
    eidx = jax.lax.broadcasted_iota(jnp.int32, (BT, E), 1)
    v1 = jnp.max(p, axis=1, keepdims=True)
    i1 = jnp.min(jnp.where(p == v1, eidx, E), axis=1, keepdims=True)
    oh1 = eidx == i1
    p2 = jnp.where(oh1, -1.0, p)
    v2 = jnp.max(p2, axis=1, keepdims=True)
    i2 = jnp.min(jnp.where(p2 == v2, eidx, E), axis=1, keepdims=True)
    oh2 = eidx == i2
    denom = v1 + v2 + 1e-9
    g_ref[...] = (jnp.where(oh1, v1, 0.0) + jnp.where(oh2, v2, 0.0)) / denom
    m_ref[...] = jnp.where(oh1 | oh2, 1.0, 0.0)


def _meta_kernel(g_ref, m_ref, s0_ref, s1_ref, wa_ref, wb_ref,
                 te_ref, tot_ref):
    mf = m_ref[...]
    g = g_ref[...]
    acc = mf
    s = 1
    while s < T:
        acc = acc + jnp.concatenate(
            [jnp.zeros((s, E), jnp.float32), acc[:T - s, :]], axis=0)
        s *= 2
    rank = acc - mf
    counts = acc[T - 1:T, :]
    ci = counts.astype(jnp.int32)
    pcount = ((ci + (BTILE - 1)) // BTILE) * BTILE
    inc = pcount
    s = 1
    while s < E:
        inc = inc + jnp.concatenate(
            [jnp.zeros((1, s), jnp.int32), inc[:, :E - s]], axis=1)
        s *= 2
    base = inc - pcount
    total = inc[:, E - 1:E]
    slot = base.astype(jnp.float32) + rank
    eidx = jax.lax.broadcasted_iota(jnp.int32, (T, E), 1)
    ea = jnp.min(jnp.where(mf > 0, eidx, E), axis=1, keepdims=True)
    eb = jnp.max(jnp.where(mf > 0, eidx, -1), axis=1, keepdims=True)
    oha = eidx == ea
    ohb = eidx == eb
    s0_ref[...] = jnp.sum(jnp.where(oha, slot, 0.0), axis=1,
                          keepdims=True).astype(jnp.int32)
    s1_ref[...] = jnp.sum(jnp.where(ohb, slot, 0.0), axis=1,
                          keepdims=True).astype(jnp.int32)
    wa_ref[...] = jnp.sum(jnp.where(oha, g, 0.0), axis=1, keepdims=True)
    wb_ref[...] = jnp.sum(jnp.where(ohb, g, 0.0), axis=1, keepdims=True)
    jj = jax.lax.broadcasted_iota(jnp.int32, (NT, E), 0) * BTILE
    pos = jnp.minimum(jj, total - 1)
    cmp = (pos >= base).astype(jnp.int32)
    te_ref[...] = jnp.sum(cmp, axis=1, keepdims=True) - 1
    tot_ref[...] = total


def _ffn_kernel(te_ref, tot_ref, buf_ref, wg_ref, wu_ref, wd_ref, eo_ref):
    j = pl.program_id(0)

    @pl.when(j * BTILE < tot_ref[0])
    def _body():
        h = buf_ref[...]
        a = jnp.dot(h, wg_ref[0], preferred_element_type=jnp.float32)
        b = jnp.dot(h, wu_ref[0], preferred_element_type=jnp.float32)
        he = (a * jax.nn.sigmoid(a)) * b
        eo_ref[...] = jnp.dot(he, wd_ref[0],
                              preferred_element_type=jnp.float32)


def _combine_kernel(x2_ref, ra_ref, rb_ref, wa_ref, wb_ref, out_ref):
    out_ref[...] = x2_ref[...] + wa_ref[...] * ra_ref[...] \
        + wb_ref[...] * rb_ref[...]


_SC_CORES = 2
_SC_SUBCORES = 16
_NW = _SC_CORES * _SC_SUBCORES
_TPW = T // _NW
_SC_MESH = plsc.VectorSubcoreMesh(core_axis_name="c", subcore_axis_name="s")


@functools.partial(
    pl.kernel, mesh=_SC_MESH,
    out_type=jax.ShapeDtypeStruct((CAP, DIM), jnp.float32),
    scratch_types=[
        pltpu.VMEM((_TPW,), jnp.int32),
        pltpu.VMEM((_TPW,), jnp.int32),
        pltpu.VMEM((_TPW, DIM), jnp.float32),
        pltpu.SemaphoreType.DMA,
    ],
)
def _sc_dispatch(h2_hbm, s0_hbm, s1_hbm, buf_hbm, s0_v, s1_v, rows_v, sem):
    wid = lax.axis_index("s") * _SC_CORES + lax.axis_index("c")
    base = wid * _TPW
    pltpu.sync_copy(s0_hbm.at[pl.ds(base, _TPW)], s0_v)
    pltpu.sync_copy(s1_hbm.at[pl.ds(base, _TPW)], s1_v)
    pltpu.sync_copy(h2_hbm.at[pl.ds(base, _TPW)], rows_v)
    pltpu.async_copy(rows_v, buf_hbm.at[s0_v], sem).wait()
    pltpu.async_copy(rows_v, buf_hbm.at[s1_v], sem).wait()


@functools.partial(
    pl.kernel, mesh=_SC_MESH,
    out_type=[
        jax.ShapeDtypeStruct((T, DIM), jnp.float32),
        jax.ShapeDtypeStruct((T, DIM), jnp.float32),
    ],
    scratch_types=[
        pltpu.VMEM((_TPW,), jnp.int32),
        pltpu.VMEM((_TPW, DIM), jnp.float32),
        pltpu.SemaphoreType.DMA,
    ],
)
def _sc_gather(eo_hbm, s0_hbm, s1_hbm, ra_hbm, rb_hbm, idx_v, rows_v, sem):
    wid = lax.axis_index("s") * _SC_CORES + lax.axis_index("c")
    base = wid * _TPW
    pltpu.sync_copy(s0_hbm.at[pl.ds(base, _TPW)], idx_v)
    pltpu.async_copy(eo_hbm.at[idx_v], rows_v, sem).wait()
    pltpu.sync_copy(rows_v, ra_hbm.at[pl.ds(base, _TPW)])
    pltpu.sync_copy(s1_hbm.at[pl.ds(base, _TPW)], idx_v)
    pltpu.async_copy(eo_hbm.at[idx_v], rows_v, sem).wait()
    pltpu.sync_copy(rows_v, rb_hbm.at[pl.ds(base, _TPW)])


def _dispatch(h2, s0, s1):
    return _sc_dispatch(h2, s0, s1)


def _gather(eo, s0, s1):
    return _sc_gather(eo, s0, s1)


def kernel(x, cos, sin, norm_attn_w, Wq, Wk, Wv, Wo, norm_ffn_w, Wr, br,
           Wg, Wu, Wd):
    xf = x.reshape(T, DIM)
    wq = jnp.take(Wq, _QPERM, axis=1)
    wk = jnp.take(Wk, _KPERM, axis=1)
    cq = jnp.tile(cos, (1, QW // (HD // 2)))
    shead = jnp.concatenate([-sin, sin], axis=1)
    sq = jnp.tile(shead, (1, H))
    ck = jnp.tile(cos, (1, KW // (HD // 2)))
    sk = jnp.tile(shead, (1, KV))
    naw = norm_attn_w.reshape(1, DIM)
    nfw = norm_ffn_w.reshape(1, DIM)
    brr = br.reshape(1, E)

    nb = T // BT
    q, k, v = pl.pallas_call(
        _qkv_kernel,
        grid=(nb,),
        in_specs=[
            pl.BlockSpec((BT, DIM), lambda i: (i, 0)),
            pl.BlockSpec((DIM, QW), lambda i: (0, 0)),
            pl.BlockSpec((DIM, KW), lambda i: (0, 0)),
            pl.BlockSpec((DIM, KW), lambda i: (0, 0)),
            pl.BlockSpec((1, DIM), lambda i: (0, 0)),
            pl.BlockSpec((BT, QW), lambda i: (i, 0)),
            pl.BlockSpec((BT, QW), lambda i: (i, 0)),
            pl.BlockSpec((BT, KW), lambda i: (i, 0)),
            pl.BlockSpec((BT, KW), lambda i: (i, 0)),
        ],
        out_specs=[
            pl.BlockSpec((BT, QW), lambda i: (i, 0)),
            pl.BlockSpec((BT, KW), lambda i: (i, 0)),
            pl.BlockSpec((BT, KW), lambda i: (i, 0)),
        ],
        out_shape=[
            jax.ShapeDtypeStruct((T, QW), jnp.float32),
            jax.ShapeDtypeStruct((T, KW), jnp.float32),
            jax.ShapeDtypeStruct((T, KW), jnp.float32),
        ],
    )(xf, wq, wk, Wv, naw, cq, sq, ck, sk)

    o = pl.pallas_call(
        _attn_kernel,
        grid=(nb,),
        in_specs=[
            pl.BlockSpec((BT, QW), lambda i: (i, 0)),
            pl.BlockSpec((T, KW), lambda i: (0, 0)),
            pl.BlockSpec((T, KW), lambda i: (0, 0)),
        ],
        out_specs=pl.BlockSpec((BT, QW), lambda i: (i, 0)),
        out_shape=jax.ShapeDtypeStruct((T, QW), jnp.float32),
    )(q, k, v)

    x2, h2, gates, mask = pl.pallas_call(
        _oproj_router_kernel,
        grid=(nb,),
        in_specs=[
            pl.BlockSpec((BT, DIM), lambda i: (i, 0)),
            pl.BlockSpec((BT, QW), lambda i: (i, 0)),
            pl.BlockSpec((QW, DIM), lambda i: (0, 0)),
            pl.BlockSpec((1, DIM), lambda i: (0, 0)),
            pl.BlockSpec((DIM, E), lambda i: (0, 0)),
            pl.BlockSpec((1, E), lambda i: (0, 0)),
        ],
        out_specs=[
            pl.BlockSpec((BT, DIM), lambda i: (i, 0)),
            pl.BlockSpec((BT, DIM), lambda i: (i, 0)),
            pl.BlockSpec((BT, E), lambda i: (i, 0)),
            pl.BlockSpec((BT, E), lambda i: (i, 0)),
        ],
        out_shape=[
            jax.ShapeDtypeStruct((T, DIM), jnp.float32),
            jax.ShapeDtypeStruct((T, DIM), jnp.float32),
            jax.ShapeDtypeStruct((T, E), jnp.float32),
            jax.ShapeDtypeStruct((T, E), jnp.float32),
        ],
    )(xf, o, Wo, nfw, Wr, brr)

    s0, s1, wa, wb, texp, tot = pl.pallas_call(
        _meta_kernel,
        out_shape=[
            jax.ShapeDtypeStruct((T, 1), jnp.int32),
            jax.ShapeDtypeStruct((T, 1), jnp.int32),
            jax.ShapeDtypeStruct((T, 1), jnp.float32),
            jax.ShapeDtypeStruct((T, 1), jnp.float32),
            jax.ShapeDtypeStruct((NT, 1), jnp.int32),
            jax.ShapeDtypeStruct((1, 1), jnp.int32),
        ],
    )(gates, mask)

    s0f = s0.reshape(T)
    s1f = s1.reshape(T)
    buf = _dispatch(h2, s0f, s1f)

    eo = pl.pallas_call(
        _ffn_kernel,
        grid_spec=pltpu.PrefetchScalarGridSpec(
            num_scalar_prefetch=2,
            grid=(NT,),
            in_specs=[
                pl.BlockSpec((BTILE, DIM), lambda j, te, to: (j, 0)),
                pl.BlockSpec((1, DIM, HID), lambda j, te, to: (te[j], 0, 0)),
                pl.BlockSpec((1, DIM, HID), lambda j, te, to: (te[j], 0, 0)),
                pl.BlockSpec((1, HID, DIM), lambda j, te, to: (te[j], 0, 0)),
            ],
            out_specs=pl.BlockSpec((BTILE, DIM), lambda j, te, to: (j, 0)),
        ),
        out_shape=jax.ShapeDtypeStruct((CAP, DIM), jnp.float32),
        compiler_params=pltpu.CompilerParams(
            dimension_semantics=("arbitrary",),
        ),
    )(texp.reshape(NT), tot.reshape(1), buf, Wg, Wu, Wd)

    ra, rb = _gather(eo, s0f, s1f)

    out = pl.pallas_call(
        _combine_kernel,
        grid=(nb,),
        in_specs=[
            pl.BlockSpec((BT, DIM), lambda i: (i, 0)),
            pl.BlockSpec((BT, DIM), lambda i: (i, 0)),
            pl.BlockSpec((BT, DIM), lambda i: (i, 0)),
            pl.BlockSpec((BT, 1), lambda i: (i, 0)),
            pl.BlockSpec((BT, 1), lambda i: (i, 0)),
        ],
        out_specs=pl.BlockSpec((BT, DIM), lambda i: (i, 0)),
        out_shape=jax.ShapeDtypeStruct((T, DIM), jnp.float32),
    )(x2, ra, rb, wa, wb)

    return out.reshape(1, T, DIM)


# speedup vs baseline: 1.4098x; 1.4098x over previous
import functools
import numpy as np
import jax
import jax.numpy as jnp
from jax import lax
from jax.experimental import pallas as pl
from jax.experimental.pallas import tpu as pltpu
from jax.experimental.pallas import tpu_sc as plsc

DIM = 768
H = 12
KV = 4
HD = 64
HID = 2048
E = 8
T = 2048
BT = 256
QW = H * HD
KW = KV * HD
BTILE = 256
NT = 24
CAP = NT * BTILE


def _halves_perm(nheads):
    perm = []
    for h in range(nheads):
        for i in range(HD // 2):
            perm.append(h * HD + 2 * i)
        for i in range(HD // 2):
            perm.append(h * HD + 2 * i + 1)
    return np.array(perm, dtype=np.int32)


_QPERM = _halves_perm(H)
_KPERM = _halves_perm(KV)


def _qkv_kernel(x_ref, wq_ref, wk_ref, wv_ref, nw_ref,
                cq_ref, sq_ref, ck_ref, sk_ref,
                q_ref, k_ref, v_ref):
    x = x_ref[...]
    rms = jnp.sqrt(jnp.mean(x * x, axis=-1, keepdims=True) + 1e-6)
    h = nw_ref[...] * x / rms
    q = jnp.dot(h, wq_ref[...], preferred_element_type=jnp.float32)
    k = jnp.dot(h, wk_ref[...], preferred_element_type=jnp.float32)
    v = jnp.dot(h, wv_ref[...], preferred_element_type=jnp.float32)
    HH = HD // 2
    qcol = jax.lax.broadcasted_iota(jnp.int32, (BT, QW), 1)
    rq = jnp.where((qcol // HH) % 2 == 0,
                   jnp.roll(q, -HH, axis=1), jnp.roll(q, HH, axis=1))
    kcol = jax.lax.broadcasted_iota(jnp.int32, (BT, KW), 1)
    rk = jnp.where((kcol // HH) % 2 == 0,
                   jnp.roll(k, -HH, axis=1), jnp.roll(k, HH, axis=1))
    q_ref[...] = q * cq_ref[...] + rq * sq_ref[...]
    k_ref[...] = k * ck_ref[...] + rk * sk_ref[...]
    v_ref[...] = v


def _attn_kernel(q_ref, k_ref, v_ref, o_ref):
    i = pl.program_id(0)
    q = q_ref[...]
    k = k_ref[...]
    v = v_ref[...]
    rowid = i * BT + jax.lax.broadcasted_iota(jnp.int32, (BT, T), 0)
    colid = jax.lax.broadcasted_iota(jnp.int32, (BT, T), 1)
    neg = jnp.where(colid > rowid, -jnp.inf, 0.0)
    scale = HD ** -0.5
    dn = (((1,), (1,)), ((), ()))
    for h in range(H):
        g = h // (H // KV)
        qh = q[:, h * HD:(h + 1) * HD]
        kh = k[:, g * HD:(g + 1) * HD]
        s = jax.lax.dot_general(qh, kh, dn,
                                preferred_element_type=jnp.float32)
        s = s * scale + neg
        m = jnp.max(s, axis=1, keepdims=True)
        p = jnp.exp(s - m)
        p = p / jnp.sum(p, axis=1, keepdims=True)
        o_ref[:, h * HD:(h + 1) * HD] = jnp.dot(
            p, v[:, g * HD:(g + 1) * HD], preferred_element_type=jnp.float32)


def _oproj_router_kernel(x_ref, o_ref, wo_ref, nw_ref, wr_ref, br_ref,
                         x2_ref, h2_ref, g_ref, m_ref):
    x2 = x_ref[...] + jnp.dot(o_ref[...], wo_ref[...],
                              preferred_element_type=jnp.float32)
    x2_ref[...] = x2
    rms = jnp.sqrt(jnp.mean(x2 * x2, axis=-1, keepdims=True) + 1e-6)
    h2 = nw_ref[...] * x2 / rms
    h2_ref[...] = h2
    logits = jnp.dot(h2, wr_ref[...], preferred_element_type=jnp.float32) \
        + br_ref[...]
    m = jnp.max(logits, axis=1, keepdims=True)
    p = jnp.exp(logits - m)
    p = p / jnp.sum(p, axis=1, keepdims=True)
    eidx = jax.lax.broadcasted_iota(jnp.int32, (BT, E), 1)
    v1 = jnp.max(p, axis=1, keepdims=True)
    i1 = jnp.min(jnp.where(p == v1, eidx, E), axis=1, keepdims=True)
    oh1 = eidx == i1
    p2 = jnp.where(oh1, -1.0, p)
    v2 = jnp.max(p2, axis=1, keepdims=True)
    i2 = jnp.min(jnp.where(p2 == v2, eidx, E), axis=1, keepdims=True)
    oh2 = eidx == i2
    denom = v1 + v2 + 1e-9
    g_ref[...] = (jnp.where(oh1, v1, 0.0) + jnp.where(oh2, v2, 0.0)) / denom
    m_ref[...] = jnp.where(oh1 | oh2, 1.0, 0.0)


def _meta_kernel(g_ref, m_ref, s0_ref, s1_ref, wa_ref, wb_ref,
                 te_ref, tot_ref):
    mf = m_ref[...]
    g = g_ref[...]
    acc = mf
    s = 1
    while s < T:
        acc = acc + jnp.concatenate(
            [jnp.zeros((s, E), jnp.float32), acc[:T - s, :]], axis=0)
        s *= 2
    rank = acc - mf
    counts = acc[T - 1:T, :]
    ci = counts.astype(jnp.int32)
    pcount = ((ci + (BTILE - 1)) // BTILE) * BTILE
    inc = pcount
    s = 1
    while s < E:
        inc = inc + jnp.concatenate(
            [jnp.zeros((1, s), jnp.int32), inc[:, :E - s]], axis=1)
        s *= 2
    base = inc - pcount
    total = inc[:, E - 1:E]
    slot = base.astype(jnp.float32) + rank
    eidx = jax.lax.broadcasted_iota(jnp.int32, (T, E), 1)
    ea = jnp.min(jnp.where(mf > 0, eidx, E), axis=1, keepdims=True)
    eb = jnp.max(jnp.where(mf > 0, eidx, -1), axis=1, keepdims=True)
    oha = eidx == ea
    ohb = eidx == eb
    s0_ref[...] = jnp.sum(jnp.where(oha, slot, 0.0), axis=1,
                          keepdims=True).astype(jnp.int32)
    s1_ref[...] = jnp.sum(jnp.where(ohb, slot, 0.0), axis=1,
                          keepdims=True).astype(jnp.int32)
    wa_ref[...] = jnp.sum(jnp.where(oha, g, 0.0), axis=1, keepdims=True)
    wb_ref[...] = jnp.sum(jnp.where(ohb, g, 0.0), axis=1, keepdims=True)
    jj = jax.lax.broadcasted_iota(jnp.int32, (NT, E), 0) * BTILE
    pos = jnp.minimum(jj, total - 1)
    cmp = (pos >= base).astype(jnp.int32)
    te_ref[...] = jnp.sum(cmp, axis=1, keepdims=True) - 1
    tot_ref[...] = total


def _ffn_kernel(te_ref, tot_ref, buf_ref, wg_ref, wu_ref, wd_ref, eo_ref):
    j = pl.program_id(0)

    @pl.when(j * BTILE < tot_ref[0])
    def _body():
        h = buf_ref[...]
        a = jnp.dot(h, wg_ref[0], preferred_element_type=jnp.float32)
        b = jnp.dot(h, wu_ref[0], preferred_element_type=jnp.float32)
        he = (a * jax.nn.sigmoid(a)) * b
        eo_ref[...] = jnp.dot(he, wd_ref[0],
                              preferred_element_type=jnp.float32)


def _combine_kernel(x2_ref, ra_ref, rb_ref, wa_ref, wb_ref, out_ref):
    out_ref[...] = x2_ref[...] + wa_ref[...] * ra_ref[...] \
        + wb_ref[...] * rb_ref[...]


_SC_CORES = 2
_SC_SUBCORES = 16
_NW = _SC_CORES * _SC_SUBCORES
_TPW = T // _NW
_SC_MESH = plsc.VectorSubcoreMesh(core_axis_name="c", subcore_axis_name="s")


@functools.partial(
    pl.kernel, mesh=_SC_MESH,
    out_type=jax.ShapeDtypeStruct((CAP, DIM), jnp.float32),
    scratch_types=[
        pltpu.VMEM((_TPW,), jnp.int32),
        pltpu.VMEM((_TPW,), jnp.int32),
        pltpu.VMEM((_TPW, DIM), jnp.float32),
        pltpu.SemaphoreType.DMA,
    ],
)
def _sc_dispatch(h2_hbm, s0_hbm, s1_hbm, buf_hbm, s0_v, s1_v, rows_v, sem):
    wid = lax.axis_index("s") * _SC_CORES + lax.axis_index("c")
    base = wid * _TPW
    pltpu.sync_copy(s0_hbm.at[pl.ds(base, _TPW)], s0_v)
    pltpu.sync_copy(s1_hbm.at[pl.ds(base, _TPW)], s1_v)
    pltpu.sync_copy(h2_hbm.at[pl.ds(base, _TPW)], rows_v)
    pltpu.async_copy(rows_v, buf_hbm.at[s0_v], sem).wait()
    pltpu.async_copy(rows_v, buf_hbm.at[s1_v], sem).wait()


@functools.partial(
    pl.kernel, mesh=_SC_MESH,
    out_type=[
        jax.ShapeDtypeStruct((T, DIM), jnp.float32),
        jax.ShapeDtypeStruct((T, DIM), jnp.float32),
    ],
    scratch_types=[
        pltpu.VMEM((_TPW,), jnp.int32),
        pltpu.VMEM((_TPW, DIM), jnp.float32),
        pltpu.SemaphoreType.DMA,
    ],
)
def _sc_gather(eo_hbm, s0_hbm, s1_hbm, ra_hbm, rb_hbm, idx_v, rows_v, sem):
    wid = lax.axis_index("s") * _SC_CORES + lax.axis_index("c")
    base = wid * _TPW
    pltpu.sync_copy(s0_hbm.at[pl.ds(base, _TPW)], idx_v)
    pltpu.async_copy(eo_hbm.at[idx_v], rows_v, sem).wait()
    pltpu.sync_copy(rows_v, ra_hbm.at[pl.ds(base, _TPW)])
    pltpu.sync_copy(s1_hbm.at[pl.ds(base, _TPW)], idx_v)
    pltpu.async_copy(eo_hbm.at[idx_v], rows_v, sem).wait()
    pltpu.sync_copy(rows_v, rb_hbm.at[pl.ds(base, _TPW)])


def _dispatch(h2, s0, s1):
    return _sc_dispatch(h2, s0, s1)


def _gather(eo, s0, s1):
    return _sc_gather(eo, s0, s1)


def kernel(x, cos, sin, norm_attn_w, Wq, Wk, Wv, Wo, norm_ffn_w, Wr, br,
           Wg, Wu, Wd):
    xf = x.reshape(T, DIM)
    wq = jnp.take(Wq, _QPERM, axis=1)
    wk = jnp.take(Wk, _KPERM, axis=1)
    cq = jnp.tile(cos, (1, QW // (HD // 2)))
    shead = jnp.concatenate([-sin, sin], axis=1)
    sq = jnp.tile(shead, (1, H))
    ck = jnp.tile(cos, (1, KW // (HD // 2)))
    sk = jnp.tile(shead, (1, KV))
    naw = norm_attn_w.reshape(1, DIM)
    nfw = norm_ffn_w.reshape(1, DIM)
    brr = br.reshape(1, E)

    nb = T // BT
    q, k, v = pl.pallas_call(
        _qkv_kernel,
        grid=(nb,),
        in_specs=[
            pl.BlockSpec((BT, DIM), lambda i: (i, 0)),
            pl.BlockSpec((DIM, QW), lambda i: (0, 0)),
            pl.BlockSpec((DIM, KW), lambda i: (0, 0)),
            pl.BlockSpec((DIM, KW), lambda i: (0, 0)),
            pl.BlockSpec((1, DIM), lambda i: (0, 0)),
            pl.BlockSpec((BT, QW), lambda i: (i, 0)),
            pl.BlockSpec((BT, QW), lambda i: (i, 0)),
            pl.BlockSpec((BT, KW), lambda i: (i, 0)),
            pl.BlockSpec((BT, KW), lambda i: (i, 0)),
        ],
        out_specs=[
            pl.BlockSpec((BT, QW), lambda i: (i, 0)),
            pl.BlockSpec((BT, KW), lambda i: (i, 0)),
            pl.BlockSpec((BT, KW), lambda i: (i, 0)),
        ],
        out_shape=[
            jax.ShapeDtypeStruct((T, QW), jnp.float32),
            jax.ShapeDtypeStruct((T, KW), jnp.float32),
            jax.ShapeDtypeStruct((T, KW), jnp.float32),
        ],
    )(xf, wq, wk, Wv, naw, cq, sq, ck, sk)

    o = pl.pallas_call(
        _attn_kernel,
        grid=(nb,),
        in_specs=[
            pl.BlockSpec((BT, QW), lambda i: (i, 0)),
            pl.BlockSpec((T, KW), lambda i: (0, 0)),
            pl.BlockSpec((T, KW), lambda i: (0, 0)),
        ],
        out_specs=pl.BlockSpec((BT, QW), lambda i: (i, 0)),
        out_shape=jax.ShapeDtypeStruct((T, QW), jnp.float32),
    )(q, k, v)

    x2, h2, gates, mask = pl.pallas_call(
        _oproj_router_kernel,
        grid=(nb,),
        in_specs=[
            pl.BlockSpec((BT, DIM), lambda i: (i, 0)),
            pl.BlockSpec((BT, QW), lambda i: (i, 0)),
            pl.BlockSpec((QW, DIM), lambda i: (0, 0)),
            pl.BlockSpec((1, DIM), lambda i: (0, 0)),
            pl.BlockSpec((DIM, E), lambda i: (0, 0)),
            pl.BlockSpec((1, E), lambda i: (0, 0)),
        ],
        out_specs=[
            pl.BlockSpec((BT, DIM), lambda i: (i, 0)),
            pl.BlockSpec((BT, DIM), lambda i: (i, 0)),
            pl.BlockSpec((BT, E), lambda i: (i, 0)),
            pl.BlockSpec((BT, E), lambda i: (i, 0)),
        ],
        out_shape=[
            jax.ShapeDtypeStruct((T, DIM), jnp.float32),
            jax.ShapeDtypeStruct((T, DIM), jnp.float32),
            jax.ShapeDtypeStruct((T, E), jnp.float32),
            jax.ShapeDtypeStruct((T, E), jnp.float32),
        ],
    )(xf, o, Wo, nfw, Wr, brr)

    s0, s1, wa, wb, texp, tot = pl.pallas_call(
        _meta_kernel,
        out_shape=[
            jax.ShapeDtypeStruct((T, 1), jnp.int32),
            jax.ShapeDtypeStruct((T, 1), jnp.int32),
            jax.ShapeDtypeStruct((T, 1), jnp.float32),
            jax.ShapeDtypeStruct((T, 1), jnp.float32),
            jax.ShapeDtypeStruct((NT, 1), jnp.int32),
            jax.ShapeDtypeStruct((1, 1), jnp.int32),
        ],
    )(gates, mask)

    s0f = s0.reshape(T)
    s1f = s1.reshape(T)
    buf = _dispatch(h2, s0f, s1f)

    eo = pl.pallas_call(
        _ffn_kernel,
        grid_spec=pltpu.PrefetchScalarGridSpec(
            num_scalar_prefetch=2,
            grid=(NT,),
            in_specs=[
                pl.BlockSpec((BTILE, DIM), lambda j, te, to: (j, 0)),
                pl.BlockSpec((1, DIM, HID), lambda j, te, to: (te[j], 0, 0)),
                pl.BlockSpec((1, DIM, HID), lambda j, te, to: (te[j], 0, 0)),
                pl.BlockSpec((1, HID, DIM), lambda j, te, to: (te[j], 0, 0)),
            ],
            out_specs=pl.BlockSpec((BTILE, DIM), lambda j, te, to: (j, 0)),
        ),
        out_shape=jax.ShapeDtypeStruct((CAP, DIM), jnp.float32),
        compiler_params=pltpu.CompilerParams(
            dimension_semantics=("arbitrary",),
        ),
    )(texp.reshape(NT), tot.reshape(1), buf, Wg, Wu, Wd)

    ra, rb = _gather(eo, s0f, s1f)

    out = pl.pallas_call(
        _combine_kernel,
        grid=(nb,),
        in_specs=[
            pl.BlockSpec((BT, DIM), lambda i: (i, 0)),
            pl.BlockSpec((BT, DIM), lambda i: (i, 0)),
            pl.BlockSpec((BT, DIM), lambda i: (i, 0)),
            pl.BlockSpec((BT, 1), lambda i: (i, 0)),
            pl.BlockSpec((BT, 1), lambda i: (i, 0)),
        ],
        out_specs=pl.BlockSpec((BT, DIM), lambda i: (i, 0)),
        out_shape=jax.ShapeDtypeStruct((T, DIM), jnp.float32),
    )(x2, ra, rb, wa, wb)

    return out.reshape(1, T, DIM)


# speedup vs baseline: 1.4932x; 1.0591x over previous
import functools
import numpy as np
import jax
import jax.numpy as jnp
from jax import lax
from jax.experimental import pallas as pl
from jax.experimental.pallas import tpu as pltpu
from jax.experimental.pallas import tpu_sc as plsc

DIM = 768
H = 12
KV = 4
HD = 64
HID = 2048
E = 8
T = 2048
BT = 256
QW = H * HD
KW = KV * HD
BTILE = 256
NT = 24
CAP = NT * BTILE


def _qkv_kernel(x_ref, wq_ref, wk_ref, wv_ref, nw_ref,
                cq_ref, sq_ref, ck_ref, sk_ref,
                q_ref, k_ref, v_ref):
    x = x_ref[...]
    rms = jnp.sqrt(jnp.mean(x * x, axis=-1, keepdims=True) + 1e-6)
    h = nw_ref[...] * x / rms
    q = jnp.dot(h, wq_ref[...], preferred_element_type=jnp.float32)
    k = jnp.dot(h, wk_ref[...], preferred_element_type=jnp.float32)
    v = jnp.dot(h, wv_ref[...], preferred_element_type=jnp.float32)
    qcol = jax.lax.broadcasted_iota(jnp.int32, (BT, QW), 1)
    rq = jnp.where(qcol % 2 == 0,
                   jnp.roll(q, -1, axis=1), jnp.roll(q, 1, axis=1))
    kcol = jax.lax.broadcasted_iota(jnp.int32, (BT, KW), 1)
    rk = jnp.where(kcol % 2 == 0,
                   jnp.roll(k, -1, axis=1), jnp.roll(k, 1, axis=1))
    q_ref[...] = q * cq_ref[...] + rq * sq_ref[...]
    k_ref[...] = k * ck_ref[...] + rk * sk_ref[...]
    v_ref[...] = v


def _attn_kernel(q_ref, k_ref, v_ref, o_ref, *, tk, off):
    i = pl.program_id(0)
    q = q_ref[...]
    k = k_ref[...]
    v = v_ref[...]
    rowid = (i + off) * BT + jax.lax.broadcasted_iota(jnp.int32, (BT, tk), 0)
    colid = jax.lax.broadcasted_iota(jnp.int32, (BT, tk), 1)
    neg = jnp.where(colid > rowid, -jnp.inf, 0.0)
    scale = HD ** -0.5
    dn = (((1,), (1,)), ((), ()))
    for h in range(H):
        g = h // (H // KV)
        qh = q[:, h * HD:(h + 1) * HD]
        kh = k[:, g * HD:(g + 1) * HD]
        s = jax.lax.dot_general(qh, kh, dn,
                                preferred_element_type=jnp.float32)
        s = s * scale + neg
        m = jnp.max(s, axis=1, keepdims=True)
        p = jnp.exp(s - m)
        p = p / jnp.sum(p, axis=1, keepdims=True)
        o_ref[:, h * HD:(h + 1) * HD] = jnp.dot(
            p, v[:, g * HD:(g + 1) * HD], preferred_element_type=jnp.float32)


def _oproj_router_kernel(x_ref, o_ref, wo_ref, nw_ref, wr_ref, br_ref,
                         x2_ref, h2_ref, g_ref, m_ref):
    x2 = x_ref[...] + jnp.dot(o_ref[...], wo_ref[...],
                              preferred_element_type=jnp.float32)
    x2_ref[...] = x2
    rms = jnp.sqrt(jnp.mean(x2 * x2, axis=-1, keepdims=True) + 1e-6)
    h2 = nw_ref[...] * x2 / rms
    h2_ref[...] = h2
    logits = jnp.dot(h2, wr_ref[...], preferred_element_type=jnp.float32) \
        + br_ref[...]
    m = jnp.max(logits, axis=1, keepdims=True)
    p = jnp.exp(logits - m)
    p = p / jnp.sum(p, axis=1, keepdims=True)
    eidx = jax.lax.broadcasted_iota(jnp.int32, (BT, E), 1)
    v1 = jnp.max(p, axis=1, keepdims=True)
    i1 = jnp.min(jnp.where(p == v1, eidx, E), axis=1, keepdims=True)
    oh1 = eidx == i1
    p2 = jnp.where(oh1, -1.0, p)
    v2 = jnp.max(p2, axis=1, keepdims=True)
    i2 = jnp.min(jnp.where(p2 == v2, eidx, E), axis=1, keepdims=True)
    oh2 = eidx == i2
    denom = v1 + v2 + 1e-9
    g_ref[...] = (jnp.where(oh1, v1, 0.0) + jnp.where(oh2, v2, 0.0)) / denom
    m_ref[...] = jnp.where(oh1 | oh2, 1.0, 0.0)


def _meta_kernel(g_ref, m_ref, s0_ref, s1_ref, wa_ref, wb_ref,
                 te_ref, tot_ref):
    mf = m_ref[...]
    g = g_ref[...]
    acc = mf
    s = 1
    while s < T:
        acc = acc + jnp.concatenate(
            [jnp.zeros((s, E), jnp.float32), acc[:T - s, :]], axis=0)
        s *= 2
    rank = acc - mf
    counts = acc[T - 1:T, :]
    ci = counts.astype(jnp.int32)
    pcount = ((ci + (BTILE - 1)) // BTILE) * BTILE
    inc = pcount
    s = 1
    while s < E:
        inc = inc + jnp.concatenate(
            [jnp.zeros((1, s), jnp.int32), inc[:, :E - s]], axis=1)
        s *= 2
    base = inc - pcount
    total = inc[:, E - 1:E]
    slot = base.astype(jnp.float32) + rank
    eidx = jax.lax.broadcasted_iota(jnp.int32, (T, E), 1)
    ea = jnp.min(jnp.where(mf > 0, eidx, E), axis=1, keepdims=True)
    eb = jnp.max(jnp.where(mf > 0, eidx, -1), axis=1, keepdims=True)
    oha = eidx == ea
    ohb = eidx == eb
    s0_ref[...] = jnp.sum(jnp.where(oha, slot, 0.0), axis=1,
                          keepdims=True).astype(jnp.int32)
    s1_ref[...] = jnp.sum(jnp.where(ohb, slot, 0.0), axis=1,
                          keepdims=True).astype(jnp.int32)
    wa_ref[...] = jnp.sum(jnp.where(oha, g, 0.0), axis=1, keepdims=True)
    wb_ref[...] = jnp.sum(jnp.where(ohb, g, 0.0), axis=1, keepdims=True)
    jj = jax.lax.broadcasted_iota(jnp.int32, (NT, E), 0) * BTILE
    pos = jnp.minimum(jj, total - 1)
    cmp = (pos >= base).astype(jnp.int32)
    te_ref[...] = jnp.sum(cmp, axis=1, keepdims=True) - 1
    tot_ref[...] = total


def _ffn_kernel(te_ref, tot_ref, buf_ref, wg_ref, wu_ref, wd_ref, eo_ref):
    j = pl.program_id(0)

    @pl.when(j * BTILE < tot_ref[0])
    def _body():
        h = buf_ref[...]
        a = jnp.dot(h, wg_ref[0], preferred_element_type=jnp.float32)
        b = jnp.dot(h, wu_ref[0], preferred_element_type=jnp.float32)
        he = (a * jax.nn.sigmoid(a)) * b
        eo_ref[...] = jnp.dot(he, wd_ref[0],
                              preferred_element_type=jnp.float32)


def _combine_kernel(x2_ref, ra_ref, rb_ref, wa_ref, wb_ref, out_ref):
    out_ref[...] = x2_ref[...] + wa_ref[...] * ra_ref[...] \
        + wb_ref[...] * rb_ref[...]


_SC_CORES = 2
_SC_SUBCORES = 16
_NW = _SC_CORES * _SC_SUBCORES
_TPW = T // _NW
_SC_MESH = plsc.VectorSubcoreMesh(core_axis_name="c", subcore_axis_name="s")


@functools.partial(
    pl.kernel, mesh=_SC_MESH,
    out_type=jax.ShapeDtypeStruct((CAP, DIM), jnp.float32),
    scratch_types=[
        pltpu.VMEM((_TPW,), jnp.int32),
        pltpu.VMEM((_TPW,), jnp.int32),
        pltpu.VMEM((_TPW, DIM), jnp.float32),
        pltpu.SemaphoreType.DMA,
    ],
)
def _sc_dispatch(h2_hbm, s0_hbm, s1_hbm, buf_hbm, s0_v, s1_v, rows_v, sem):
    wid = lax.axis_index("s") * _SC_CORES + lax.axis_index("c")
    base = wid * _TPW
    pltpu.sync_copy(s0_hbm.at[pl.ds(base, _TPW)], s0_v)
    pltpu.sync_copy(s1_hbm.at[pl.ds(base, _TPW)], s1_v)
    pltpu.sync_copy(h2_hbm.at[pl.ds(base, _TPW)], rows_v)
    pltpu.async_copy(rows_v, buf_hbm.at[s0_v], sem).wait()
    pltpu.async_copy(rows_v, buf_hbm.at[s1_v], sem).wait()


@functools.partial(
    pl.kernel, mesh=_SC_MESH,
    out_type=[
        jax.ShapeDtypeStruct((T, DIM), jnp.float32),
        jax.ShapeDtypeStruct((T, DIM), jnp.float32),
    ],
    scratch_types=[
        pltpu.VMEM((_TPW,), jnp.int32),
        pltpu.VMEM((_TPW, DIM), jnp.float32),
        pltpu.SemaphoreType.DMA,
    ],
)
def _sc_gather(eo_hbm, s0_hbm, s1_hbm, ra_hbm, rb_hbm, idx_v, rows_v, sem):
    wid = lax.axis_index("s") * _SC_CORES + lax.axis_index("c")
    base = wid * _TPW
    pltpu.sync_copy(s0_hbm.at[pl.ds(base, _TPW)], idx_v)
    pltpu.async_copy(eo_hbm.at[idx_v], rows_v, sem).wait()
    pltpu.sync_copy(rows_v, ra_hbm.at[pl.ds(base, _TPW)])
    pltpu.sync_copy(s1_hbm.at[pl.ds(base, _TPW)], idx_v)
    pltpu.async_copy(eo_hbm.at[idx_v], rows_v, sem).wait()
    pltpu.sync_copy(rows_v, rb_hbm.at[pl.ds(base, _TPW)])


def _dispatch(h2, s0, s1):
    return _sc_dispatch(h2, s0, s1)


def _gather(eo, s0, s1):
    return _sc_gather(eo, s0, s1)


def kernel(x, cos, sin, norm_attn_w, Wq, Wk, Wv, Wo, norm_ffn_w, Wr, br,
           Wg, Wu, Wd):
    xf = x.reshape(T, DIM)
    c64 = jnp.repeat(cos, 2, axis=1)
    s64 = jnp.stack([-sin, sin], axis=2).reshape(T, HD)
    cq = jnp.tile(c64, (1, H))
    sq = jnp.tile(s64, (1, H))
    ck = jnp.tile(c64, (1, KV))
    sk = jnp.tile(s64, (1, KV))
    naw = norm_attn_w.reshape(1, DIM)
    nfw = norm_ffn_w.reshape(1, DIM)
    brr = br.reshape(1, E)

    nb = T // BT
    q, k, v = pl.pallas_call(
        _qkv_kernel,
        grid=(nb,),
        in_specs=[
            pl.BlockSpec((BT, DIM), lambda i: (i, 0)),
            pl.BlockSpec((DIM, QW), lambda i: (0, 0)),
            pl.BlockSpec((DIM, KW), lambda i: (0, 0)),
            pl.BlockSpec((DIM, KW), lambda i: (0, 0)),
            pl.BlockSpec((1, DIM), lambda i: (0, 0)),
            pl.BlockSpec((BT, QW), lambda i: (i, 0)),
            pl.BlockSpec((BT, QW), lambda i: (i, 0)),
            pl.BlockSpec((BT, KW), lambda i: (i, 0)),
            pl.BlockSpec((BT, KW), lambda i: (i, 0)),
        ],
        out_specs=[
            pl.BlockSpec((BT, QW), lambda i: (i, 0)),
            pl.BlockSpec((BT, KW), lambda i: (i, 0)),
            pl.BlockSpec((BT, KW), lambda i: (i, 0)),
        ],
        out_shape=[
            jax.ShapeDtypeStruct((T, QW), jnp.float32),
            jax.ShapeDtypeStruct((T, KW), jnp.float32),
            jax.ShapeDtypeStruct((T, KW), jnp.float32),
        ],
    )(xf, Wq, Wk, Wv, naw, cq, sq, ck, sk)

    TA = T // 2
    oa = pl.pallas_call(
        functools.partial(_attn_kernel, tk=TA, off=0),
        grid=(nb // 2,),
        in_specs=[
            pl.BlockSpec((BT, QW), lambda i: (i, 0)),
            pl.BlockSpec((TA, KW), lambda i: (0, 0)),
            pl.BlockSpec((TA, KW), lambda i: (0, 0)),
        ],
        out_specs=pl.BlockSpec((BT, QW), lambda i: (i, 0)),
        out_shape=jax.ShapeDtypeStruct((TA, QW), jnp.float32),
    )(q, k[:TA], v[:TA])
    ob = pl.pallas_call(
        functools.partial(_attn_kernel, tk=T, off=nb // 2),
        grid=(nb // 2,),
        in_specs=[
            pl.BlockSpec((BT, QW), lambda i: (i + nb // 2, 0)),
            pl.BlockSpec((T, KW), lambda i: (0, 0)),
            pl.BlockSpec((T, KW), lambda i: (0, 0)),
        ],
        out_specs=pl.BlockSpec((BT, QW), lambda i: (i, 0)),
        out_shape=jax.ShapeDtypeStruct((TA, QW), jnp.float32),
    )(q, k, v)
    o = jnp.concatenate([oa, ob], axis=0)

    x2, h2, gates, mask = pl.pallas_call(
        _oproj_router_kernel,
        grid=(nb,),
        in_specs=[
            pl.BlockSpec((BT, DIM), lambda i: (i, 0)),
            pl.BlockSpec((BT, QW), lambda i: (i, 0)),
            pl.BlockSpec((QW, DIM), lambda i: (0, 0)),
            pl.BlockSpec((1, DIM), lambda i: (0, 0)),
            pl.BlockSpec((DIM, E), lambda i: (0, 0)),
            pl.BlockSpec((1, E), lambda i: (0, 0)),
        ],
        out_specs=[
            pl.BlockSpec((BT, DIM), lambda i: (i, 0)),
            pl.BlockSpec((BT, DIM), lambda i: (i, 0)),
            pl.BlockSpec((BT, E), lambda i: (i, 0)),
            pl.BlockSpec((BT, E), lambda i: (i, 0)),
        ],
        out_shape=[
            jax.ShapeDtypeStruct((T, DIM), jnp.float32),
            jax.ShapeDtypeStruct((T, DIM), jnp.float32),
            jax.ShapeDtypeStruct((T, E), jnp.float32),
            jax.ShapeDtypeStruct((T, E), jnp.float32),
        ],
    )(xf, o, Wo, nfw, Wr, brr)

    s0, s1, wa, wb, texp, tot = pl.pallas_call(
        _meta_kernel,
        out_shape=[
            jax.ShapeDtypeStruct((T, 1), jnp.int32),
            jax.ShapeDtypeStruct((T, 1), jnp.int32),
            jax.ShapeDtypeStruct((T, 1), jnp.float32),
            jax.ShapeDtypeStruct((T, 1), jnp.float32),
            jax.ShapeDtypeStruct((NT, 1), jnp.int32),
            jax.ShapeDtypeStruct((1, 1), jnp.int32),
        ],
    )(gates, mask)

    s0f = s0.reshape(T)
    s1f = s1.reshape(T)
    buf = _dispatch(h2, s0f, s1f)

    eo = pl.pallas_call(
        _ffn_kernel,
        grid_spec=pltpu.PrefetchScalarGridSpec(
            num_scalar_prefetch=2,
            grid=(NT,),
            in_specs=[
                pl.BlockSpec((BTILE, DIM), lambda j, te, to: (j, 0)),
                pl.BlockSpec((1, DIM, HID), lambda j, te, to: (te[j], 0, 0)),
                pl.BlockSpec((1, DIM, HID), lambda j, te, to: (te[j], 0, 0)),
                pl.BlockSpec((1, HID, DIM), lambda j, te, to: (te[j], 0, 0)),
            ],
            out_specs=pl.BlockSpec((BTILE, DIM), lambda j, te, to: (j, 0)),
        ),
        out_shape=jax.ShapeDtypeStruct((CAP, DIM), jnp.float32),
        compiler_params=pltpu.CompilerParams(
            dimension_semantics=("arbitrary",),
        ),
    )(texp.reshape(NT), tot.reshape(1), buf, Wg, Wu, Wd)

    ra, rb = _gather(eo, s0f, s1f)

    out = pl.pallas_call(
        _combine_kernel,
        grid=(nb,),
        in_specs=[
            pl.BlockSpec((BT, DIM), lambda i: (i, 0)),
            pl.BlockSpec((BT, DIM), lambda i: (i, 0)),
            pl.BlockSpec((BT, DIM), lambda i: (i, 0)),
            pl.BlockSpec((BT, 1), lambda i: (i, 0)),
            pl.BlockSpec((BT, 1), lambda i: (i, 0)),
        ],
        out_specs=pl.BlockSpec((BT, DIM), lambda i: (i, 0)),
        out_shape=jax.ShapeDtypeStruct((T, DIM), jnp.float32),
    )(x2, ra, rb, wa, wb)

    return out.reshape(1, T, DIM)


# speedup vs baseline: 1.6625x; 1.1134x over previous
import functools
import numpy as np
import jax
import jax.numpy as jnp
from jax import lax
from jax.experimental import pallas as pl
from jax.experimental.pallas import tpu as pltpu
from jax.experimental.pallas import tpu_sc as plsc

DIM = 768
H = 12
KV = 4
HD = 64
HID = 2048
E = 8
T = 2048
BT = 256
QW = H * HD
KW = KV * HD
BTILE = 256
NT = 24
CAP = NT * BTILE


def _qkv_kernel(x_ref, wq_ref, wk_ref, wv_ref, nw_ref,
                cq_ref, sq_ref,
                q_ref, k_ref, v_ref):
    x = x_ref[...]
    rms = jnp.sqrt(jnp.mean(x * x, axis=-1, keepdims=True) + 1e-6)
    h = nw_ref[...] * x / rms
    q = jnp.dot(h, wq_ref[...], preferred_element_type=jnp.float32)
    k = jnp.dot(h, wk_ref[...], preferred_element_type=jnp.float32)
    v = jnp.dot(h, wv_ref[...], preferred_element_type=jnp.float32)
    cq = jnp.tile(cq_ref[...], (1, H))
    sq = jnp.tile(sq_ref[...], (1, H))
    ck = jnp.tile(cq_ref[...], (1, KV))
    sk = jnp.tile(sq_ref[...], (1, KV))
    qcol = jax.lax.broadcasted_iota(jnp.int32, (BT, QW), 1)
    rq = jnp.where(qcol % 2 == 0,
                   jnp.roll(q, -1, axis=1), jnp.roll(q, 1, axis=1))
    kcol = jax.lax.broadcasted_iota(jnp.int32, (BT, KW), 1)
    rk = jnp.where(kcol % 2 == 0,
                   jnp.roll(k, -1, axis=1), jnp.roll(k, 1, axis=1))
    q_ref[...] = q * cq + rq * sq
    k_ref[...] = k * ck + rk * sk
    v_ref[...] = v


def _attn_kernel(q_ref, k_ref, v_ref, o_ref, *, tk, off):
    i = pl.program_id(0)
    q = q_ref[...]
    k = k_ref[...]
    v = v_ref[...]
    rowid = (i + off) * BT + jax.lax.broadcasted_iota(jnp.int32, (BT, tk), 0)
    colid = jax.lax.broadcasted_iota(jnp.int32, (BT, tk), 1)
    neg = jnp.where(colid > rowid, -jnp.inf, 0.0)
    scale = HD ** -0.5
    dn = (((1,), (1,)), ((), ()))
    for h in range(H):
        g = h // (H // KV)
        qh = q[:, h * HD:(h + 1) * HD]
        kh = k[:, g * HD:(g + 1) * HD]
        s = jax.lax.dot_general(qh, kh, dn,
                                preferred_element_type=jnp.float32)
        s = s * scale + neg
        m = jnp.max(s, axis=1, keepdims=True)
        p = jnp.exp(s - m)
        p = p / jnp.sum(p, axis=1, keepdims=True)
        o_ref[:, h * HD:(h + 1) * HD] = jnp.dot(
            p, v[:, g * HD:(g + 1) * HD], preferred_element_type=jnp.float32)


def _oproj_router_meta_kernel(x_ref, o_ref, wo_ref, nw_ref, wr_ref, br_ref,
                              x2_ref, h2_ref, s0_ref, s1_ref, wa_ref, wb_ref,
                              te_ref, tot_ref):
    x2 = x_ref[...] + jnp.dot(o_ref[...], wo_ref[...],
                              preferred_element_type=jnp.float32)
    x2_ref[...] = x2
    rms = jnp.sqrt(jnp.mean(x2 * x2, axis=-1, keepdims=True) + 1e-6)
    h2 = nw_ref[...] * x2 / rms
    h2_ref[...] = h2
    logits = jnp.dot(h2, wr_ref[...], preferred_element_type=jnp.float32) \
        + br_ref[...]
    m = jnp.max(logits, axis=1, keepdims=True)
    p = jnp.exp(logits - m)
    p = p / jnp.sum(p, axis=1, keepdims=True)
    eidx = jax.lax.broadcasted_iota(jnp.int32, (T, E), 1)
    v1 = jnp.max(p, axis=1, keepdims=True)
    i1 = jnp.min(jnp.where(p == v1, eidx, E), axis=1, keepdims=True)
    oh1 = eidx == i1
    p2 = jnp.where(oh1, -1.0, p)
    v2 = jnp.max(p2, axis=1, keepdims=True)
    i2 = jnp.min(jnp.where(p2 == v2, eidx, E), axis=1, keepdims=True)
    oh2 = eidx == i2
    denom = v1 + v2 + 1e-9
    g = (jnp.where(oh1, v1, 0.0) + jnp.where(oh2, v2, 0.0)) / denom
    mf = jnp.where(oh1 | oh2, 1.0, 0.0)
    acc = mf
    s = 1
    while s < T:
        acc = acc + jnp.concatenate(
            [jnp.zeros((s, E), jnp.float32), acc[:T - s, :]], axis=0)
        s *= 2
    rank = acc - mf
    counts = acc[T - 1:T, :]
    ci = counts.astype(jnp.int32)
    pcount = ((ci + (BTILE - 1)) // BTILE) * BTILE
    inc = pcount
    s = 1
    while s < E:
        inc = inc + jnp.concatenate(
            [jnp.zeros((1, s), jnp.int32), inc[:, :E - s]], axis=1)
        s *= 2
    base = inc - pcount
    total = inc[:, E - 1:E]
    slot = base.astype(jnp.float32) + rank
    eidx = jax.lax.broadcasted_iota(jnp.int32, (T, E), 1)
    ea = jnp.min(jnp.where(mf > 0, eidx, E), axis=1, keepdims=True)
    eb = jnp.max(jnp.where(mf > 0, eidx, -1), axis=1, keepdims=True)
    oha = eidx == ea
    ohb = eidx == eb
    s0_ref[...] = jnp.sum(jnp.where(oha, slot, 0.0), axis=1,
                          keepdims=True).astype(jnp.int32)
    s1_ref[...] = jnp.sum(jnp.where(ohb, slot, 0.0), axis=1,
                          keepdims=True).astype(jnp.int32)
    wa_ref[...] = jnp.sum(jnp.where(oha, g, 0.0), axis=1, keepdims=True)
    wb_ref[...] = jnp.sum(jnp.where(ohb, g, 0.0), axis=1, keepdims=True)
    jj = jax.lax.broadcasted_iota(jnp.int32, (NT, E), 0) * BTILE
    pos = jnp.minimum(jj, total - 1)
    cmp = (pos >= base).astype(jnp.int32)
    te_ref[...] = jnp.sum(cmp, axis=1, keepdims=True) - 1
    tot_ref[...] = total


def _ffn_kernel(te_ref, tot_ref, buf_ref, wg_ref, wu_ref, wd_ref, eo_ref):
    j = pl.program_id(0)

    @pl.when(j * BTILE < tot_ref[0])
    def _body():
        h = buf_ref[...]
        a = jnp.dot(h, wg_ref[0], preferred_element_type=jnp.float32)
        b = jnp.dot(h, wu_ref[0], preferred_element_type=jnp.float32)
        he = (a * jax.nn.sigmoid(a)) * b
        eo_ref[...] = jnp.dot(he, wd_ref[0],
                              preferred_element_type=jnp.float32)


def _combine_kernel(x2_ref, ra_ref, rb_ref, wa_ref, wb_ref, out_ref):
    out_ref[...] = x2_ref[...] + wa_ref[...] * ra_ref[...] \
        + wb_ref[...] * rb_ref[...]


_SC_CORES = 2
_SC_SUBCORES = 16
_NW = _SC_CORES * _SC_SUBCORES
_TPW = T // _NW
_SC_MESH = plsc.VectorSubcoreMesh(core_axis_name="c", subcore_axis_name="s")


@functools.partial(
    pl.kernel, mesh=_SC_MESH,
    out_type=jax.ShapeDtypeStruct((CAP, DIM), jnp.float32),
    scratch_types=[
        pltpu.VMEM((_TPW,), jnp.int32),
        pltpu.VMEM((_TPW,), jnp.int32),
        pltpu.VMEM((_TPW, DIM), jnp.float32),
        pltpu.SemaphoreType.DMA,
    ],
)
def _sc_dispatch(h2_hbm, s0_hbm, s1_hbm, buf_hbm, s0_v, s1_v, rows_v, sem):
    wid = lax.axis_index("s") * _SC_CORES + lax.axis_index("c")
    base = wid * _TPW
    pltpu.sync_copy(s0_hbm.at[pl.ds(base, _TPW)], s0_v)
    pltpu.sync_copy(s1_hbm.at[pl.ds(base, _TPW)], s1_v)
    pltpu.sync_copy(h2_hbm.at[pl.ds(base, _TPW)], rows_v)
    pltpu.async_copy(rows_v, buf_hbm.at[s0_v], sem).wait()
    pltpu.async_copy(rows_v, buf_hbm.at[s1_v], sem).wait()


@functools.partial(
    pl.kernel, mesh=_SC_MESH,
    out_type=[
        jax.ShapeDtypeStruct((T, DIM), jnp.float32),
        jax.ShapeDtypeStruct((T, DIM), jnp.float32),
    ],
    scratch_types=[
        pltpu.VMEM((_TPW,), jnp.int32),
        pltpu.VMEM((_TPW, DIM), jnp.float32),
        pltpu.SemaphoreType.DMA,
    ],
)
def _sc_gather(eo_hbm, s0_hbm, s1_hbm, ra_hbm, rb_hbm, idx_v, rows_v, sem):
    wid = lax.axis_index("s") * _SC_CORES + lax.axis_index("c")
    base = wid * _TPW
    pltpu.sync_copy(s0_hbm.at[pl.ds(base, _TPW)], idx_v)
    pltpu.async_copy(eo_hbm.at[idx_v], rows_v, sem).wait()
    pltpu.sync_copy(rows_v, ra_hbm.at[pl.ds(base, _TPW)])
    pltpu.sync_copy(s1_hbm.at[pl.ds(base, _TPW)], idx_v)
    pltpu.async_copy(eo_hbm.at[idx_v], rows_v, sem).wait()
    pltpu.sync_copy(rows_v, rb_hbm.at[pl.ds(base, _TPW)])


def _dispatch(h2, s0, s1):
    return _sc_dispatch(h2, s0, s1)


def _gather(eo, s0, s1):
    return _sc_gather(eo, s0, s1)


def kernel(x, cos, sin, norm_attn_w, Wq, Wk, Wv, Wo, norm_ffn_w, Wr, br,
           Wg, Wu, Wd):
    xf = x.reshape(T, DIM)
    c64 = jnp.repeat(cos, 2, axis=1)
    s64 = jnp.stack([-sin, sin], axis=2).reshape(T, HD)
    naw = norm_attn_w.reshape(1, DIM)
    nfw = norm_ffn_w.reshape(1, DIM)
    brr = br.reshape(1, E)

    nb = T // BT
    q, k, v = pl.pallas_call(
        _qkv_kernel,
        grid=(nb,),
        in_specs=[
            pl.BlockSpec((BT, DIM), lambda i: (i, 0)),
            pl.BlockSpec((DIM, QW), lambda i: (0, 0)),
            pl.BlockSpec((DIM, KW), lambda i: (0, 0)),
            pl.BlockSpec((DIM, KW), lambda i: (0, 0)),
            pl.BlockSpec((1, DIM), lambda i: (0, 0)),
            pl.BlockSpec((BT, HD), lambda i: (i, 0)),
            pl.BlockSpec((BT, HD), lambda i: (i, 0)),
        ],
        out_specs=[
            pl.BlockSpec((BT, QW), lambda i: (i, 0)),
            pl.BlockSpec((BT, KW), lambda i: (i, 0)),
            pl.BlockSpec((BT, KW), lambda i: (i, 0)),
        ],
        out_shape=[
            jax.ShapeDtypeStruct((T, QW), jnp.float32),
            jax.ShapeDtypeStruct((T, KW), jnp.float32),
            jax.ShapeDtypeStruct((T, KW), jnp.float32),
        ],
    )(xf, Wq, Wk, Wv, naw, c64, s64)

    TA = T // 2
    oa = pl.pallas_call(
        functools.partial(_attn_kernel, tk=TA, off=0),
        grid=(nb // 2,),
        in_specs=[
            pl.BlockSpec((BT, QW), lambda i: (i, 0)),
            pl.BlockSpec((TA, KW), lambda i: (0, 0)),
            pl.BlockSpec((TA, KW), lambda i: (0, 0)),
        ],
        out_specs=pl.BlockSpec((BT, QW), lambda i: (i, 0)),
        out_shape=jax.ShapeDtypeStruct((TA, QW), jnp.float32),
    )(q, k[:TA], v[:TA])
    ob = pl.pallas_call(
        functools.partial(_attn_kernel, tk=T, off=nb // 2),
        grid=(nb // 2,),
        in_specs=[
            pl.BlockSpec((BT, QW), lambda i: (i + nb // 2, 0)),
            pl.BlockSpec((T, KW), lambda i: (0, 0)),
            pl.BlockSpec((T, KW), lambda i: (0, 0)),
        ],
        out_specs=pl.BlockSpec((BT, QW), lambda i: (i, 0)),
        out_shape=jax.ShapeDtypeStruct((TA, QW), jnp.float32),
    )(q, k, v)
    o = jnp.concatenate([oa, ob], axis=0)

    x2, h2, s0, s1, wa, wb, texp, tot = pl.pallas_call(
        _oproj_router_meta_kernel,
        out_shape=[
            jax.ShapeDtypeStruct((T, DIM), jnp.float32),
            jax.ShapeDtypeStruct((T, DIM), jnp.float32),
            jax.ShapeDtypeStruct((T, 1), jnp.int32),
            jax.ShapeDtypeStruct((T, 1), jnp.int32),
            jax.ShapeDtypeStruct((T, 1), jnp.float32),
            jax.ShapeDtypeStruct((T, 1), jnp.float32),
            jax.ShapeDtypeStruct((NT, 1), jnp.int32),
            jax.ShapeDtypeStruct((1, 1), jnp.int32),
        ],
    )(xf, o, Wo, nfw, Wr, brr)

    s0f = s0.reshape(T)
    s1f = s1.reshape(T)
    buf = _dispatch(h2, s0f, s1f)

    eo = pl.pallas_call(
        _ffn_kernel,
        grid_spec=pltpu.PrefetchScalarGridSpec(
            num_scalar_prefetch=2,
            grid=(NT,),
            in_specs=[
                pl.BlockSpec((BTILE, DIM), lambda j, te, to: (j, 0)),
                pl.BlockSpec((1, DIM, HID), lambda j, te, to: (te[j], 0, 0)),
                pl.BlockSpec((1, DIM, HID), lambda j, te, to: (te[j], 0, 0)),
                pl.BlockSpec((1, HID, DIM), lambda j, te, to: (te[j], 0, 0)),
            ],
            out_specs=pl.BlockSpec((BTILE, DIM), lambda j, te, to: (j, 0)),
        ),
        out_shape=jax.ShapeDtypeStruct((CAP, DIM), jnp.float32),
        compiler_params=pltpu.CompilerParams(
            dimension_semantics=("arbitrary",),
        ),
    )(texp.reshape(NT), tot.reshape(1), buf, Wg, Wu, Wd)

    ra, rb = _gather(eo, s0f, s1f)

    out = pl.pallas_call(
        _combine_kernel,
        grid=(nb,),
        in_specs=[
            pl.BlockSpec((BT, DIM), lambda i: (i, 0)),
            pl.BlockSpec((BT, DIM), lambda i: (i, 0)),
            pl.BlockSpec((BT, DIM), lambda i: (i, 0)),
            pl.BlockSpec((BT, 1), lambda i: (i, 0)),
            pl.BlockSpec((BT, 1), lambda i: (i, 0)),
        ],
        out_specs=pl.BlockSpec((BT, DIM), lambda i: (i, 0)),
        out_shape=jax.ShapeDtypeStruct((T, DIM), jnp.float32),
    )(x2, ra, rb, wa, wb)

    return out.reshape(1, T, DIM)
